# baseline (speedup 1.0000x reference)
import functools

import jax
import jax.numpy as jnp
from jax import lax
from jax.experimental import pallas as pl
from jax.experimental.pallas import tpu as pltpu
from jax.experimental.pallas import tpu_sc as plsc

N = 10000
C = 128
E = 320000
NC = 2
NS = 16
NW = NC * NS
B = 128
K = 80
EPT = K * B
E_PAD = NW * EPT
N_PAD = 10112
RPT = N_PAD // NS
CPT = C // NW
FL = CPT * N_PAD
CH = 1024
NPAIR = E_PAD // (2 * CH)

_MESH = dict(
    mesh=plsc.VectorSubcoreMesh(
        core_axis_name="c", subcore_axis_name="s", num_cores=NC, num_subcores=NS
    )
)


@functools.partial(
    pl.kernel,
    out_type=jax.ShapeDtypeStruct((NC, N_PAD, C), jnp.float32),
    scratch_types=[
        pltpu.VMEM((K, B), jnp.int32),
        pltpu.VMEM((B, C), jnp.float32),
        pltpu.VMEM_SHARED((N_PAD, C), jnp.float32),
    ],
    **_MESH,
)
def _deg(dst_hbm, zeros_hbm, out_hbm, idx_ref, ones_ref, acc_ref):
    cid = lax.axis_index("c")
    sid = lax.axis_index("s")
    wid = sid * NC + cid
    rows = pl.ds(sid * RPT, RPT)
    pltpu.sync_copy(zeros_hbm.at[rows], acc_ref.at[rows])
    pltpu.sync_copy(dst_hbm.at[wid], idx_ref)
    one = jnp.full((16,), 1.0, jnp.float32)
    for r in range(B):
        for cc in range(C // 16):
            ones_ref[r, pl.ds(cc * 16, 16)] = one
    plsc.subcore_barrier()
    for g in range(K):
        pltpu.sync_copy(ones_ref, acc_ref.at[idx_ref.at[g]], add=True)
    plsc.subcore_barrier()
    pltpu.sync_copy(acc_ref.at[rows], out_hbm.at[cid, rows])


@functools.partial(
    pl.kernel,
    out_type=jax.ShapeDtypeStruct((C * N_PAD,), jnp.float32),
    scratch_types=[
        pltpu.VMEM((FL,), jnp.float32),
        pltpu.VMEM((FL,), jnp.float32),
        pltpu.VMEM((CH,), jnp.int32),
        pltpu.VMEM((CH,), jnp.int32),
        pltpu.VMEM((CH,), jnp.int32),
        pltpu.VMEM((CH,), jnp.int32),
        pltpu.SemaphoreType.DMA,
        pltpu.SemaphoreType.DMA,
        pltpu.SemaphoreType.DMA,
        pltpu.SemaphoreType.DMA,
    ],
    compiler_params=pltpu.CompilerParams(needs_layout_passes=False),
    **_MESH,
)
def _agg(hsT_hbm, src_hbm, dst_hbm, zeros_hbm, out_hbm,
         hs_ref, acc_ref, src0, dst0, src1, dst1, ss0, sd0, ss1, sd1):
    cid = lax.axis_index("c")
    sid = lax.axis_index("s")
    wid = sid * NC + cid
    mine = pl.ds(wid * FL, FL)
    pltpu.sync_copy(hsT_hbm.at[mine], hs_ref)
    pltpu.sync_copy(zeros_hbm.at[pl.ds(0, FL)], acc_ref)

    def chunk(idx_s, idx_d):
        for j in range(CH // 16):
            s16 = idx_s[pl.ds(j * 16, 16)]
            d16 = idx_d[pl.ds(j * 16, 16)]
            for c in range(CPT):
                v = plsc.load_gather(hs_ref, (s16 + (c * N_PAD),))
                plsc.addupdate_scatter(acc_ref, (d16 + (c * N_PAD),), v)

    pltpu.async_copy(src_hbm.at[pl.ds(0, CH)], src0, ss0)
    pltpu.async_copy(dst_hbm.at[pl.ds(0, CH)], dst0, sd0)

    def body(t, carry):
        base = t * (2 * CH)
        pltpu.async_copy(src_hbm.at[pl.ds(base + CH, CH)], src1, ss1)
        pltpu.async_copy(dst_hbm.at[pl.ds(base + CH, CH)], dst1, sd1)
        pltpu.make_async_copy(src_hbm.at[pl.ds(base, CH)], src0, ss0).wait()
        pltpu.make_async_copy(dst_hbm.at[pl.ds(base, CH)], dst0, sd0).wait()
        chunk(src0, dst0)

        @pl.when(t + 1 < NPAIR)
        def _prefetch():
            pltpu.async_copy(src_hbm.at[pl.ds(base + 2 * CH, CH)], src0, ss0)
            pltpu.async_copy(dst_hbm.at[pl.ds(base + 2 * CH, CH)], dst0, sd0)

        pltpu.make_async_copy(src_hbm.at[pl.ds(base + CH, CH)], src1, ss1).wait()
        pltpu.make_async_copy(dst_hbm.at[pl.ds(base + CH, CH)], dst1, sd1).wait()
        chunk(src1, dst1)
        return carry

    lax.fori_loop(0, NPAIR, body, 0)
    pltpu.sync_copy(acc_ref, out_hbm.at[mine])


_RB = 128
_GRID = N_PAD // _RB


def _dis_of(degp_ref):
    d = degp_ref[0, :, 0] + degp_ref[1, :, 0] + 1.0
    return lax.rsqrt(d)[:, None]


def _k1_body(x_ref, w_ref, degp_ref, o_ref, ot_ref):
    h = jnp.dot(x_ref[...], w_ref[...], preferred_element_type=jnp.float32)
    hs = h * _dis_of(degp_ref)
    o_ref[...] = hs
    ot_ref[...] = hs.T


def _k2_body(pt_ref, hs_ref, degp_ref, b_ref, a_ref, w_ref, o_ref, ot_ref):
    dis = _dis_of(degp_ref)
    y = (pt_ref[...].T + hs_ref[...]) * dis + b_ref[...]
    h = jnp.where(y >= 0.0, y, a_ref[...] * y)
    hs = jnp.dot(h, w_ref[...], preferred_element_type=jnp.float32) * dis
    o_ref[...] = hs
    ot_ref[...] = hs.T


def _k3_body(pt_ref, hs_ref, degp_ref, b_ref, a_ref, o_ref):
    y = (pt_ref[...].T + hs_ref[...]) * _dis_of(degp_ref) + b_ref[...]
    o_ref[...] = jnp.where(y >= 0.0, y, a_ref[...] * y)


_row_spec = pl.BlockSpec((_RB, C), lambda i: (i, 0))
_t_spec = pl.BlockSpec((C, _RB), lambda i: (0, i))
_w_spec = pl.BlockSpec((C, C), lambda i: (0, 0))
_vec_spec = pl.BlockSpec((1, C), lambda i: (0, 0))
_degp_spec = pl.BlockSpec((NC, _RB, C), lambda i: (0, i, 0))
_out_shape = jax.ShapeDtypeStruct((N_PAD, C), jnp.float32)
_outt_shape = jax.ShapeDtypeStruct((C, N_PAD), jnp.float32)

_k1 = pl.pallas_call(
    _k1_body,
    grid=(_GRID,),
    in_specs=[_row_spec, _w_spec, _degp_spec],
    out_specs=[_row_spec, _t_spec],
    out_shape=[_out_shape, _outt_shape],
)

_k2 = pl.pallas_call(
    _k2_body,
    grid=(_GRID,),
    in_specs=[_t_spec, _row_spec, _degp_spec, _vec_spec, _vec_spec, _w_spec],
    out_specs=[_row_spec, _t_spec],
    out_shape=[_out_shape, _outt_shape],
)

_k3 = pl.pallas_call(
    _k3_body,
    grid=(_GRID,),
    in_specs=[_t_spec, _row_spec, _degp_spec, _vec_spec, _vec_spec],
    out_specs=_row_spec,
    out_shape=_out_shape,
)


@jax.jit
def kernel(x, edge_index, W1, b1, alpha1, W2, b2, alpha2):
    src = edge_index[0].astype(jnp.int32)
    dst = edge_index[1].astype(jnp.int32)
    npad = E_PAD - E
    src = jnp.concatenate([src, jnp.zeros((npad,), jnp.int32)])
    dst = jnp.concatenate([dst, jnp.full((npad,), N, jnp.int32)])
    dst_r = dst.reshape(NW, K, B)

    zeros_acc = jnp.zeros((N_PAD, C), jnp.float32)
    zeros_fl = zeros_acc.reshape(-1)
    b1r = b1.reshape(1, C)
    b2r = b2.reshape(1, C)
    a1r = alpha1.reshape(1, C)
    a2r = alpha2.reshape(1, C)

    xp = jnp.pad(x, ((0, N_PAD - N), (0, 0)))
    degp = _deg(dst_r, zeros_acc)
    hs1, hs1t = _k1(xp, W1, degp)
    p1t = _agg(hs1t.reshape(-1), src, dst, zeros_fl)
    hs2, hs2t = _k2(p1t.reshape(C, N_PAD), hs1, degp, b1r, a1r, W2)
    p2t = _agg(hs2t.reshape(-1), src, dst, zeros_fl)
    return _k3(p2t.reshape(C, N_PAD), hs2, degp, b2r, a2r)[:N]


# speedup vs baseline: 1.3799x; 1.3799x over previous
import functools

import jax
import jax.numpy as jnp
from jax import lax
from jax.experimental import pallas as pl
from jax.experimental.pallas import tpu as pltpu
from jax.experimental.pallas import tpu_sc as plsc

N = 10000
C = 128
E = 320000
NC = 2
NS = 16
NW = NC * NS
B = 128
K = 80
EPT = K * B
E_PAD = NW * EPT
BG = 128
KG = EPT // BG
GT = 16
RING = 2
N_PAD = 10112
RPT = N_PAD // NS

_MESH = dict(
    mesh=plsc.VectorSubcoreMesh(
        core_axis_name="c", subcore_axis_name="s", num_cores=NC, num_subcores=NS
    )
)


@functools.partial(
    pl.kernel,
    out_type=jax.ShapeDtypeStruct((NC, N_PAD, C), jnp.float32),
    scratch_types=[
        pltpu.VMEM((K, B), jnp.int32),
        pltpu.VMEM((B, C), jnp.float32),
        pltpu.VMEM_SHARED((N_PAD, C), jnp.float32),
    ],
    **_MESH,
)
def _deg(dst_hbm, zeros_hbm, out_hbm, idx_ref, ones_ref, acc_ref):
    cid = lax.axis_index("c")
    sid = lax.axis_index("s")
    wid = sid * NC + cid
    rows = pl.ds(sid * RPT, RPT)
    pltpu.sync_copy(zeros_hbm.at[rows], acc_ref.at[rows])
    pltpu.sync_copy(dst_hbm.at[wid], idx_ref)
    one = jnp.full((16,), 1.0, jnp.float32)
    for r in range(B):
        for cc in range(C // 16):
            ones_ref[r, pl.ds(cc * 16, 16)] = one
    plsc.subcore_barrier()
    for g in range(K):
        pltpu.sync_copy(ones_ref, acc_ref.at[idx_ref.at[g]], add=True)
    plsc.subcore_barrier()
    pltpu.sync_copy(acc_ref.at[rows], out_hbm.at[cid, rows])


@functools.partial(
    pl.kernel,
    out_type=jax.ShapeDtypeStruct((NC, N_PAD, C), jnp.float32),
    scratch_types=[
        pltpu.VMEM((GT, BG), jnp.int32),
        pltpu.VMEM((GT, BG), jnp.int32),
        pltpu.VMEM((RING, BG, C), jnp.float32),
        pltpu.VMEM_SHARED((N_PAD, C), jnp.float32),
    ] + [pltpu.SemaphoreType.DMA] * RING,
    **_MESH,
)
def _agg(hs_hbm, src_hbm, dst_hbm, zeros_hbm, out_hbm,
         src_ref, dst_ref, rows_ref, acc_ref, *sems):
    cid = lax.axis_index("c")
    sid = lax.axis_index("s")
    wid = sid * NC + cid
    rows = pl.ds(sid * RPT, RPT)
    pltpu.sync_copy(zeros_hbm.at[rows], acc_ref.at[rows])
    plsc.subcore_barrier()
    for t in range(KG // GT):
        pltpu.sync_copy(src_hbm.at[wid, pl.ds(t * GT, GT)], src_ref)
        pltpu.sync_copy(dst_hbm.at[wid, pl.ds(t * GT, GT)], dst_ref)
        for w in range(RING - 1):
            pltpu.async_copy(hs_hbm.at[src_ref.at[w]], rows_ref.at[w], sems[w])
        for g in range(GT):
            buf = g % RING
            pre = g + RING - 1
            if pre < GT:
                pltpu.async_copy(
                    hs_hbm.at[src_ref.at[pre]],
                    rows_ref.at[pre % RING],
                    sems[pre % RING],
                )
            pltpu.make_async_copy(
                hs_hbm.at[src_ref.at[g]], rows_ref.at[buf], sems[buf]
            ).wait()
            pltpu.sync_copy(rows_ref.at[buf], acc_ref.at[dst_ref.at[g]], add=True)
    plsc.subcore_barrier()
    pltpu.sync_copy(acc_ref.at[rows], out_hbm.at[cid, rows])


_RB = 1000
_GRID = N // _RB


def _dis_of(degp_ref):
    d = degp_ref[0, :, 0] + degp_ref[1, :, 0] + 1.0
    return lax.rsqrt(d)[:, None]


def _k1_body(x_ref, w_ref, degp_ref, o_ref):
    h = jnp.dot(x_ref[...], w_ref[...], preferred_element_type=jnp.float32)
    o_ref[...] = h * _dis_of(degp_ref)


def _k2_body(p_ref, hs_ref, degp_ref, b_ref, a_ref, w_ref, o_ref):
    dis = _dis_of(degp_ref)
    y = (p_ref[0] + p_ref[1] + hs_ref[...]) * dis + b_ref[...]
    h = jnp.where(y >= 0.0, y, a_ref[...] * y)
    o_ref[...] = jnp.dot(h, w_ref[...], preferred_element_type=jnp.float32) * dis


def _k3_body(p_ref, hs_ref, degp_ref, b_ref, a_ref, o_ref):
    y = (p_ref[0] + p_ref[1] + hs_ref[...]) * _dis_of(degp_ref) + b_ref[...]
    o_ref[...] = jnp.where(y >= 0.0, y, a_ref[...] * y)


_row_spec = pl.BlockSpec((_RB, C), lambda i: (i, 0))
_w_spec = pl.BlockSpec((C, C), lambda i: (0, 0))
_vec_spec = pl.BlockSpec((1, C), lambda i: (0, 0))
_degp_spec = pl.BlockSpec((NC, _RB, C), lambda i: (0, i, 0))
_p_spec = pl.BlockSpec((NC, _RB, C), lambda i: (0, i, 0))
_out_shape = jax.ShapeDtypeStruct((N, C), jnp.float32)

_k1 = pl.pallas_call(
    _k1_body,
    grid=(_GRID,),
    in_specs=[_row_spec, _w_spec, _degp_spec],
    out_specs=_row_spec,
    out_shape=_out_shape,
)

_k2 = pl.pallas_call(
    _k2_body,
    grid=(_GRID,),
    in_specs=[_p_spec, _row_spec, _degp_spec, _vec_spec, _vec_spec, _w_spec],
    out_specs=_row_spec,
    out_shape=_out_shape,
)

_k3 = pl.pallas_call(
    _k3_body,
    grid=(_GRID,),
    in_specs=[_p_spec, _row_spec, _degp_spec, _vec_spec, _vec_spec],
    out_specs=_row_spec,
    out_shape=_out_shape,
)


@jax.jit
def kernel(x, edge_index, W1, b1, alpha1, W2, b2, alpha2):
    src = edge_index[0].astype(jnp.int32)
    dst = edge_index[1].astype(jnp.int32)
    npad = E_PAD - E
    src = jnp.concatenate([src, jnp.zeros((npad,), jnp.int32)])
    dst = jnp.concatenate([dst, jnp.full((npad,), N, jnp.int32)])
    src_r = src.reshape(NW, KG, BG)
    dst_r = dst.reshape(NW, K, B)
    dst_g = dst.reshape(NW, KG, BG)

    zeros_acc = jnp.zeros((N_PAD, C), jnp.float32)
    b1r = b1.reshape(1, C)
    b2r = b2.reshape(1, C)
    a1r = alpha1.reshape(1, C)
    a2r = alpha2.reshape(1, C)

    degp = _deg(dst_r, zeros_acc)
    hs1 = _k1(x, W1, degp)
    p1 = _agg(hs1, src_r, dst_g, zeros_acc)
    hs2 = _k2(p1, hs1, degp, b1r, a1r, W2)
    p2 = _agg(hs2, src_r, dst_g, zeros_acc)
    return _k3(p2, hs2, degp, b2r, a2r)


# speedup vs baseline: 1.3915x; 1.0084x over previous
import functools

import jax
import jax.numpy as jnp
from jax import lax
from jax.experimental import pallas as pl
from jax.experimental.pallas import tpu as pltpu
from jax.experimental.pallas import tpu_sc as plsc

N = 10000
C = 128
E = 320000
NC = 2
NS = 16
NW = NC * NS
B = 128
K = 80
EPT = K * B
E_PAD = NW * EPT
BG = 128
KG = EPT // BG
GT = 16
RING = 2
N_PAD = 10112
RPT = N_PAD // NS

_MESH = dict(
    mesh=plsc.VectorSubcoreMesh(
        core_axis_name="c", subcore_axis_name="s", num_cores=NC, num_subcores=NS
    )
)


@functools.partial(
    pl.kernel,
    out_type=jax.ShapeDtypeStruct((NC, N_PAD, C), jnp.float32),
    scratch_types=[
        pltpu.VMEM((K, B), jnp.int32),
        pltpu.VMEM((B, C), jnp.float32),
        pltpu.VMEM_SHARED((N_PAD, C), jnp.float32),
    ],
    **_MESH,
)
def _deg(dst_hbm, zeros_hbm, out_hbm, idx_ref, ones_ref, acc_ref):
    cid = lax.axis_index("c")
    sid = lax.axis_index("s")
    wid = sid * NC + cid
    rows = pl.ds(sid * RPT, RPT)
    pltpu.sync_copy(zeros_hbm.at[rows], acc_ref.at[rows])
    pltpu.sync_copy(dst_hbm.at[wid], idx_ref)
    one = jnp.full((16,), 1.0, jnp.float32)
    for r in range(B):
        for cc in range(C // 16):
            ones_ref[r, pl.ds(cc * 16, 16)] = one
    plsc.subcore_barrier()
    for g in range(K):
        pltpu.sync_copy(ones_ref, acc_ref.at[idx_ref.at[g]], add=True)
    plsc.subcore_barrier()
    pltpu.sync_copy(acc_ref.at[rows], out_hbm.at[cid, rows])


@functools.partial(
    pl.kernel,
    out_type=jax.ShapeDtypeStruct((NC, N_PAD, C), jnp.float32),
    scratch_types=[
        pltpu.VMEM((GT, BG), jnp.int32),
        pltpu.VMEM((KG, BG), jnp.int32),
        pltpu.VMEM((RING, BG, C), jnp.float32),
        pltpu.VMEM_SHARED((N_PAD, C), jnp.float32),
    ] + [pltpu.SemaphoreType.DMA] * RING,
    **_MESH,
)
def _agg(hs_hbm, src_hbm, dst_hbm, zeros_hbm, out_hbm,
         src_ref, dst_ref, rows_ref, acc_ref, *sems):
    cid = lax.axis_index("c")
    sid = lax.axis_index("s")
    wid = sid * NC + cid
    rows = pl.ds(sid * RPT, RPT)
    pltpu.sync_copy(zeros_hbm.at[rows], acc_ref.at[rows])
    pltpu.sync_copy(dst_hbm.at[wid], dst_ref)
    plsc.subcore_barrier()
    for t in range(KG // GT):
        pltpu.sync_copy(src_hbm.at[wid, pl.ds(t * GT, GT)], src_ref)
        for w in range(RING - 1):
            pltpu.async_copy(hs_hbm.at[src_ref.at[w]], rows_ref.at[w], sems[w])
        for g in range(GT):
            buf = g % RING
            pre = g + RING - 1
            if pre < GT:
                pltpu.async_copy(
                    hs_hbm.at[src_ref.at[pre]],
                    rows_ref.at[pre % RING],
                    sems[pre % RING],
                )
            pltpu.make_async_copy(
                hs_hbm.at[src_ref.at[g]], rows_ref.at[buf], sems[buf]
            ).wait()
            pltpu.sync_copy(
                rows_ref.at[buf], acc_ref.at[dst_ref.at[t * GT + g]], add=True
            )
    plsc.subcore_barrier()
    pltpu.sync_copy(acc_ref.at[rows], out_hbm.at[cid, rows])


_RB = 1000
_GRID = N // _RB


def _dis_of(degp_ref):
    d = degp_ref[0, :, 0] + degp_ref[1, :, 0] + 1.0
    return lax.rsqrt(d)[:, None]


def _k1_body(x_ref, w_ref, degp_ref, o_ref):
    h = jnp.dot(x_ref[...], w_ref[...], preferred_element_type=jnp.float32)
    o_ref[...] = h * _dis_of(degp_ref)


def _k2_body(p_ref, hs_ref, degp_ref, b_ref, a_ref, w_ref, o_ref):
    dis = _dis_of(degp_ref)
    y = (p_ref[0] + p_ref[1] + hs_ref[...]) * dis + b_ref[...]
    h = jnp.where(y >= 0.0, y, a_ref[...] * y)
    o_ref[...] = jnp.dot(h, w_ref[...], preferred_element_type=jnp.float32) * dis


def _k3_body(p_ref, hs_ref, degp_ref, b_ref, a_ref, o_ref):
    y = (p_ref[0] + p_ref[1] + hs_ref[...]) * _dis_of(degp_ref) + b_ref[...]
    o_ref[...] = jnp.where(y >= 0.0, y, a_ref[...] * y)


_row_spec = pl.BlockSpec((_RB, C), lambda i: (i, 0))
_w_spec = pl.BlockSpec((C, C), lambda i: (0, 0))
_vec_spec = pl.BlockSpec((1, C), lambda i: (0, 0))
_degp_spec = pl.BlockSpec((NC, _RB, C), lambda i: (0, i, 0))
_p_spec = pl.BlockSpec((NC, _RB, C), lambda i: (0, i, 0))
_out_shape = jax.ShapeDtypeStruct((N, C), jnp.float32)

_k1 = pl.pallas_call(
    _k1_body,
    grid=(_GRID,),
    in_specs=[_row_spec, _w_spec, _degp_spec],
    out_specs=_row_spec,
    out_shape=_out_shape,
)

_k2 = pl.pallas_call(
    _k2_body,
    grid=(_GRID,),
    in_specs=[_p_spec, _row_spec, _degp_spec, _vec_spec, _vec_spec, _w_spec],
    out_specs=_row_spec,
    out_shape=_out_shape,
)

_k3 = pl.pallas_call(
    _k3_body,
    grid=(_GRID,),
    in_specs=[_p_spec, _row_spec, _degp_spec, _vec_spec, _vec_spec],
    out_specs=_row_spec,
    out_shape=_out_shape,
)


@jax.jit
def kernel(x, edge_index, W1, b1, alpha1, W2, b2, alpha2):
    src = edge_index[0].astype(jnp.int32)
    dst = edge_index[1].astype(jnp.int32)
    npad = E_PAD - E
    src = jnp.concatenate([src, jnp.zeros((npad,), jnp.int32)])
    dst = jnp.concatenate([dst, jnp.full((npad,), N, jnp.int32)])
    src_r = src.reshape(NW, KG, BG)
    dst_r = dst.reshape(NW, K, B)
    dst_g = dst.reshape(NW, KG, BG)

    zeros_acc = jnp.zeros((N_PAD, C), jnp.float32)
    b1r = b1.reshape(1, C)
    b2r = b2.reshape(1, C)
    a1r = alpha1.reshape(1, C)
    a2r = alpha2.reshape(1, C)

    degp = _deg(dst_r, zeros_acc)
    hs1 = _k1(x, W1, degp)
    p1 = _agg(hs1, src_r, dst_g, zeros_acc)
    hs2 = _k2(p1, hs1, degp, b1r, a1r, W2)
    p2 = _agg(hs2, src_r, dst_g, zeros_acc)
    return _k3(p2, hs2, degp, b2r, a2r)


# speedup vs baseline: 1.4044x; 1.0093x over previous
import functools

import jax
import jax.numpy as jnp
from jax import lax
from jax.experimental import pallas as pl
from jax.experimental.pallas import tpu as pltpu
from jax.experimental.pallas import tpu_sc as plsc

N = 10000
C = 128
E = 320000
NC = 2
NS = 16
NW = NC * NS
B = 128
K = 80
EPT = K * B
E_PAD = NW * EPT
BG = 128
KG = EPT // BG
GT = 40
RING = 2
N_PAD = 10112
RPT = N_PAD // NS

_MESH = dict(
    mesh=plsc.VectorSubcoreMesh(
        core_axis_name="c", subcore_axis_name="s", num_cores=NC, num_subcores=NS
    )
)


@functools.partial(
    pl.kernel,
    out_type=jax.ShapeDtypeStruct((NC, N_PAD, C), jnp.float32),
    scratch_types=[
        pltpu.VMEM((K, B), jnp.int32),
        pltpu.VMEM((B, C), jnp.float32),
        pltpu.VMEM_SHARED((N_PAD, C), jnp.float32),
    ],
    **_MESH,
)
def _deg(dst_hbm, zeros_hbm, out_hbm, idx_ref, ones_ref, acc_ref):
    cid = lax.axis_index("c")
    sid = lax.axis_index("s")
    wid = sid * NC + cid
    rows = pl.ds(sid * RPT, RPT)
    pltpu.sync_copy(zeros_hbm.at[rows], acc_ref.at[rows])
    pltpu.sync_copy(dst_hbm.at[wid], idx_ref)
    one = jnp.full((16,), 1.0, jnp.float32)
    for r in range(B):
        for cc in range(C // 16):
            ones_ref[r, pl.ds(cc * 16, 16)] = one
    plsc.subcore_barrier()
    for g in range(K):
        pltpu.sync_copy(ones_ref, acc_ref.at[idx_ref.at[g]], add=True)
    plsc.subcore_barrier()
    pltpu.sync_copy(acc_ref.at[rows], out_hbm.at[cid, rows])


@functools.partial(
    pl.kernel,
    out_type=jax.ShapeDtypeStruct((NC, N_PAD, C), jnp.float32),
    scratch_types=[
        pltpu.VMEM((GT, BG), jnp.int32),
        pltpu.VMEM((KG, BG), jnp.int32),
        pltpu.VMEM((RING, BG, C), jnp.float32),
        pltpu.VMEM_SHARED((N_PAD, C), jnp.float32),
    ] + [pltpu.SemaphoreType.DMA] * RING,
    **_MESH,
)
def _agg(hs_hbm, src_hbm, dst_hbm, zeros_hbm, out_hbm,
         src_ref, dst_ref, rows_ref, acc_ref, *sems):
    cid = lax.axis_index("c")
    sid = lax.axis_index("s")
    wid = sid * NC + cid
    rows = pl.ds(sid * RPT, RPT)
    pltpu.sync_copy(zeros_hbm.at[rows], acc_ref.at[rows])
    pltpu.sync_copy(dst_hbm.at[wid], dst_ref)
    plsc.subcore_barrier()
    for t in range(KG // GT):
        pltpu.sync_copy(src_hbm.at[wid, pl.ds(t * GT, GT)], src_ref)
        for w in range(RING - 1):
            pltpu.async_copy(hs_hbm.at[src_ref.at[w]], rows_ref.at[w], sems[w])
        for g in range(GT):
            buf = g % RING
            pre = g + RING - 1
            if pre < GT:
                pltpu.async_copy(
                    hs_hbm.at[src_ref.at[pre]],
                    rows_ref.at[pre % RING],
                    sems[pre % RING],
                )
            pltpu.make_async_copy(
                hs_hbm.at[src_ref.at[g]], rows_ref.at[buf], sems[buf]
            ).wait()
            pltpu.sync_copy(
                rows_ref.at[buf], acc_ref.at[dst_ref.at[t * GT + g]], add=True
            )
    plsc.subcore_barrier()
    pltpu.sync_copy(acc_ref.at[rows], out_hbm.at[cid, rows])


_RB = 1000
_GRID = N // _RB


def _dis_of(degp_ref):
    d = degp_ref[0, :, 0] + degp_ref[1, :, 0] + 1.0
    return lax.rsqrt(d)[:, None]


def _k1_body(x_ref, w_ref, degp_ref, o_ref):
    h = jnp.dot(x_ref[...], w_ref[...], preferred_element_type=jnp.float32)
    o_ref[...] = h * _dis_of(degp_ref)


def _k2_body(p_ref, hs_ref, degp_ref, b_ref, a_ref, w_ref, o_ref):
    dis = _dis_of(degp_ref)
    y = (p_ref[0] + p_ref[1] + hs_ref[...]) * dis + b_ref[...]
    h = jnp.where(y >= 0.0, y, a_ref[...] * y)
    o_ref[...] = jnp.dot(h, w_ref[...], preferred_element_type=jnp.float32) * dis


def _k3_body(p_ref, hs_ref, degp_ref, b_ref, a_ref, o_ref):
    y = (p_ref[0] + p_ref[1] + hs_ref[...]) * _dis_of(degp_ref) + b_ref[...]
    o_ref[...] = jnp.where(y >= 0.0, y, a_ref[...] * y)


_row_spec = pl.BlockSpec((_RB, C), lambda i: (i, 0))
_w_spec = pl.BlockSpec((C, C), lambda i: (0, 0))
_vec_spec = pl.BlockSpec((1, C), lambda i: (0, 0))
_degp_spec = pl.BlockSpec((NC, _RB, C), lambda i: (0, i, 0))
_p_spec = pl.BlockSpec((NC, _RB, C), lambda i: (0, i, 0))
_out_shape = jax.ShapeDtypeStruct((N, C), jnp.float32)

_k1 = pl.pallas_call(
    _k1_body,
    grid=(_GRID,),
    in_specs=[_row_spec, _w_spec, _degp_spec],
    out_specs=_row_spec,
    out_shape=_out_shape,
)

_k2 = pl.pallas_call(
    _k2_body,
    grid=(_GRID,),
    in_specs=[_p_spec, _row_spec, _degp_spec, _vec_spec, _vec_spec, _w_spec],
    out_specs=_row_spec,
    out_shape=_out_shape,
)

_k3 = pl.pallas_call(
    _k3_body,
    grid=(_GRID,),
    in_specs=[_p_spec, _row_spec, _degp_spec, _vec_spec, _vec_spec],
    out_specs=_row_spec,
    out_shape=_out_shape,
)


@jax.jit
def kernel(x, edge_index, W1, b1, alpha1, W2, b2, alpha2):
    src = edge_index[0].astype(jnp.int32)
    dst = edge_index[1].astype(jnp.int32)
    npad = E_PAD - E
    src = jnp.concatenate([src, jnp.zeros((npad,), jnp.int32)])
    dst = jnp.concatenate([dst, jnp.full((npad,), N, jnp.int32)])
    src_r = src.reshape(NW, KG, BG)
    dst_r = dst.reshape(NW, K, B)
    dst_g = dst.reshape(NW, KG, BG)

    zeros_acc = jnp.zeros((N_PAD, C), jnp.float32)
    b1r = b1.reshape(1, C)
    b2r = b2.reshape(1, C)
    a1r = alpha1.reshape(1, C)
    a2r = alpha2.reshape(1, C)

    degp = _deg(dst_r, zeros_acc)
    hs1 = _k1(x, W1, degp)
    p1 = _agg(hs1, src_r, dst_g, zeros_acc)
    hs2 = _k2(p1, hs1, degp, b1r, a1r, W2)
    p2 = _agg(hs2, src_r, dst_g, zeros_acc)
    return _k3(p2, hs2, degp, b2r, a2r)


# speedup vs baseline: 1.4277x; 1.0167x over previous
import functools

import jax
import jax.numpy as jnp
from jax import lax
from jax.experimental import pallas as pl
from jax.experimental.pallas import tpu as pltpu
from jax.experimental.pallas import tpu_sc as plsc

N = 10000
C = 128
E = 320000
NC = 2
NS = 16
NW = NC * NS
B = 128
K = 80
EPT = K * B
E_PAD = NW * EPT
BG = 128
KG = EPT // BG
GT = 40
RING = 2
N_PAD = 10112
RPT = N_PAD // NS

_MESH = dict(
    mesh=plsc.VectorSubcoreMesh(
        core_axis_name="c", subcore_axis_name="s", num_cores=NC, num_subcores=NS
    )
)


@functools.partial(
    pl.kernel,
    out_type=jax.ShapeDtypeStruct((NC, N_PAD, C), jnp.float32),
    scratch_types=[
        pltpu.VMEM((K, B), jnp.int32),
        pltpu.VMEM((B, C), jnp.float32),
        pltpu.VMEM_SHARED((N_PAD, C), jnp.float32),
    ],
    **_MESH,
)
def _deg(dst_hbm, zeros_hbm, out_hbm, idx_ref, ones_ref, acc_ref):
    cid = lax.axis_index("c")
    sid = lax.axis_index("s")
    wid = sid * NC + cid
    rows = pl.ds(sid * RPT, RPT)
    pltpu.sync_copy(zeros_hbm.at[rows], acc_ref.at[rows])
    pltpu.sync_copy(dst_hbm.at[wid], idx_ref)
    one = jnp.full((16,), 1.0, jnp.float32)
    for r in range(B):
        for cc in range(C // 16):
            ones_ref[r, pl.ds(cc * 16, 16)] = one
    plsc.subcore_barrier()
    for g in range(K):
        pltpu.sync_copy(ones_ref, acc_ref.at[idx_ref.at[g]], add=True)
    plsc.subcore_barrier()
    pltpu.sync_copy(acc_ref.at[rows], out_hbm.at[cid, rows])


@functools.partial(
    pl.kernel,
    out_type=jax.ShapeDtypeStruct((NC, N_PAD, C), jnp.float32),
    scratch_types=[
        pltpu.VMEM((GT, BG), jnp.int32),
        pltpu.VMEM((KG, BG), jnp.int32),
        pltpu.VMEM((RING, BG, C), jnp.float32),
        pltpu.VMEM_SHARED((N_PAD, C), jnp.float32),
    ] + [pltpu.SemaphoreType.DMA] * RING,
    **_MESH,
)
def _agg(hs_hbm, src_hbm, dst_hbm, zeros_hbm, out_hbm,
         src_ref, dst_ref, rows_ref, acc_ref, *sems):
    cid = lax.axis_index("c")
    sid = lax.axis_index("s")
    wid = sid * NC + cid
    rows = pl.ds(sid * RPT, RPT)
    pltpu.sync_copy(zeros_hbm.at[rows], acc_ref.at[rows])
    pltpu.sync_copy(dst_hbm.at[wid], dst_ref)
    plsc.subcore_barrier()
    for t in range(KG // GT):
        pltpu.sync_copy(src_hbm.at[wid, pl.ds(t * GT, GT)], src_ref)
        for w in range(RING - 1):
            pltpu.async_copy(
                hs_hbm.at[src_ref.at[w]], rows_ref.at[w], sems[w], priority=1
            )
        for g in range(GT):
            buf = g % RING
            pre = g + RING - 1
            if pre < GT:
                pltpu.async_copy(
                    hs_hbm.at[src_ref.at[pre]],
                    rows_ref.at[pre % RING],
                    sems[pre % RING],
                    priority=1,
                )
            pltpu.make_async_copy(
                hs_hbm.at[src_ref.at[g]], rows_ref.at[buf], sems[buf]
            ).wait()
            pltpu.sync_copy(
                rows_ref.at[buf], acc_ref.at[dst_ref.at[t * GT + g]], add=True
            )
    plsc.subcore_barrier()
    pltpu.sync_copy(acc_ref.at[rows], out_hbm.at[cid, rows])


_RB = 1000
_GRID = N // _RB


def _dis_of(degp_ref):
    d = degp_ref[0, :, 0] + degp_ref[1, :, 0] + 1.0
    return lax.rsqrt(d)[:, None]


def _k1_body(x_ref, w_ref, degp_ref, o_ref):
    h = jnp.dot(x_ref[...], w_ref[...], preferred_element_type=jnp.float32)
    o_ref[...] = h * _dis_of(degp_ref)


def _k2_body(p_ref, hs_ref, degp_ref, b_ref, a_ref, w_ref, o_ref):
    dis = _dis_of(degp_ref)
    y = (p_ref[0] + p_ref[1] + hs_ref[...]) * dis + b_ref[...]
    h = jnp.where(y >= 0.0, y, a_ref[...] * y)
    o_ref[...] = jnp.dot(h, w_ref[...], preferred_element_type=jnp.float32) * dis


def _k3_body(p_ref, hs_ref, degp_ref, b_ref, a_ref, o_ref):
    y = (p_ref[0] + p_ref[1] + hs_ref[...]) * _dis_of(degp_ref) + b_ref[...]
    o_ref[...] = jnp.where(y >= 0.0, y, a_ref[...] * y)


_row_spec = pl.BlockSpec((_RB, C), lambda i: (i, 0))
_w_spec = pl.BlockSpec((C, C), lambda i: (0, 0))
_vec_spec = pl.BlockSpec((1, C), lambda i: (0, 0))
_degp_spec = pl.BlockSpec((NC, _RB, C), lambda i: (0, i, 0))
_p_spec = pl.BlockSpec((NC, _RB, C), lambda i: (0, i, 0))
_out_shape = jax.ShapeDtypeStruct((N, C), jnp.float32)

_k1 = pl.pallas_call(
    _k1_body,
    grid=(_GRID,),
    in_specs=[_row_spec, _w_spec, _degp_spec],
    out_specs=_row_spec,
    out_shape=_out_shape,
)

_k2 = pl.pallas_call(
    _k2_body,
    grid=(_GRID,),
    in_specs=[_p_spec, _row_spec, _degp_spec, _vec_spec, _vec_spec, _w_spec],
    out_specs=_row_spec,
    out_shape=_out_shape,
)

_k3 = pl.pallas_call(
    _k3_body,
    grid=(_GRID,),
    in_specs=[_p_spec, _row_spec, _degp_spec, _vec_spec, _vec_spec],
    out_specs=_row_spec,
    out_shape=_out_shape,
)


@jax.jit
def kernel(x, edge_index, W1, b1, alpha1, W2, b2, alpha2):
    src = edge_index[0].astype(jnp.int32)
    dst = edge_index[1].astype(jnp.int32)
    npad = E_PAD - E
    src = jnp.concatenate([src, jnp.zeros((npad,), jnp.int32)])
    dst = jnp.concatenate([dst, jnp.full((npad,), N, jnp.int32)])
    src_r = src.reshape(NW, KG, BG)
    dst_r = dst.reshape(NW, K, B)
    dst_g = dst.reshape(NW, KG, BG)

    zeros_acc = jnp.zeros((N_PAD, C), jnp.float32)
    b1r = b1.reshape(1, C)
    b2r = b2.reshape(1, C)
    a1r = alpha1.reshape(1, C)
    a2r = alpha2.reshape(1, C)

    degp = _deg(dst_r, zeros_acc)
    hs1 = _k1(x, W1, degp)
    p1 = _agg(hs1, src_r, dst_g, zeros_acc)
    hs2 = _k2(p1, hs1, degp, b1r, a1r, W2)
    p2 = _agg(hs2, src_r, dst_g, zeros_acc)
    return _k3(p2, hs2, degp, b2r, a2r)


# speedup vs baseline: 1.5958x; 1.1177x over previous
import functools

import jax
import jax.numpy as jnp
from jax import lax
from jax.experimental import pallas as pl
from jax.experimental.pallas import tpu as pltpu
from jax.experimental.pallas import tpu_sc as plsc

N = 10000
C = 128
E = 320000
NC = 2
NS = 16
NW = NC * NS
B = 128
K = 80
EPT = K * B
E_PAD = NW * EPT
BG = 128
KG = EPT // BG
GT = 40
RING = 2
N_PAD = 10112
RPT = N_PAD // NS

_MESH = dict(
    mesh=plsc.VectorSubcoreMesh(
        core_axis_name="c", subcore_axis_name="s", num_cores=NC, num_subcores=NS
    )
)


@functools.partial(
    pl.kernel,
    out_type=jax.ShapeDtypeStruct((NC, N_PAD, C), jnp.float32),
    scratch_types=[
        pltpu.VMEM((K, B), jnp.int32),
        pltpu.VMEM((B, C), jnp.float32),
        pltpu.VMEM_SHARED((N_PAD, C), jnp.float32),
    ],
    **_MESH,
)
def _deg(dst_hbm, zeros_hbm, out_hbm, idx_ref, ones_ref, acc_ref):
    cid = lax.axis_index("c")
    sid = lax.axis_index("s")
    wid = sid * NC + cid
    rows = pl.ds(sid * RPT, RPT)
    pltpu.sync_copy(zeros_hbm.at[rows], acc_ref.at[rows])
    pltpu.sync_copy(dst_hbm.at[wid], idx_ref)
    one = jnp.full((16,), 1.0, jnp.float32)
    for r in range(B):
        for cc in range(C // 16):
            ones_ref[r, pl.ds(cc * 16, 16)] = one
    plsc.subcore_barrier()
    for g in range(K):
        pltpu.sync_copy(ones_ref, acc_ref.at[idx_ref.at[g]], add=True)
    plsc.subcore_barrier()
    pltpu.sync_copy(acc_ref.at[rows], out_hbm.at[cid, rows])


@functools.partial(
    pl.kernel,
    out_type=jax.ShapeDtypeStruct((NC, N_PAD, C), jnp.float32),
    scratch_types=[
        pltpu.VMEM((GT, BG), jnp.int32),
        pltpu.VMEM((KG, BG), jnp.int32),
        pltpu.VMEM((RING, BG, C), jnp.float32),
        pltpu.VMEM_SHARED((N_PAD, C), jnp.float32),
    ] + [pltpu.SemaphoreType.DMA] * RING,
    **_MESH,
)
def _agg(hs_hbm, src_hbm, dst_hbm, zeros_hbm, out_hbm,
         src_ref, dst_ref, rows_ref, acc_ref, *sems):
    cid = lax.axis_index("c")
    sid = lax.axis_index("s")
    wid = sid * NC + cid
    rows = pl.ds(sid * RPT, RPT)
    pltpu.sync_copy(zeros_hbm.at[rows], acc_ref.at[rows])
    pltpu.sync_copy(dst_hbm.at[wid], dst_ref)
    plsc.subcore_barrier()
    for t in range(KG // GT):
        pltpu.sync_copy(src_hbm.at[wid, pl.ds(t * GT, GT)], src_ref)
        for w in range(RING - 1):
            pltpu.async_copy(
                hs_hbm.at[src_ref.at[w]], rows_ref.at[w], sems[w], priority=1
            )
        for g in range(GT):
            buf = g % RING
            pre = g + RING - 1
            if pre < GT:
                pltpu.async_copy(
                    hs_hbm.at[src_ref.at[pre]],
                    rows_ref.at[pre % RING],
                    sems[pre % RING],
                    priority=1,
                )
            pltpu.make_async_copy(
                hs_hbm.at[src_ref.at[g]], rows_ref.at[buf], sems[buf]
            ).wait()
            pltpu.sync_copy(
                rows_ref.at[buf], acc_ref.at[dst_ref.at[t * GT + g]], add=True
            )
    plsc.subcore_barrier()
    pltpu.sync_copy(acc_ref.at[rows], out_hbm.at[cid, rows])


_RB = 1000
_GRID = N // _RB


def _dis_of(degp_ref):
    d = degp_ref[0, :, 0] + degp_ref[1, :, 0] + 1.0
    return lax.rsqrt(d)[:, None]


def _mm_body(x_ref, w_ref, o_ref):
    o_ref[...] = jnp.dot(x_ref[...], w_ref[...], preferred_element_type=jnp.float32)


def _scale_body(h_ref, degp_ref, o_ref):
    o_ref[...] = h_ref[...] * _dis_of(degp_ref)


def _k2_body(p_ref, hs_ref, degp_ref, b_ref, a_ref, w_ref, o_ref):
    dis = _dis_of(degp_ref)
    y = (p_ref[0] + p_ref[1] + hs_ref[...]) * dis + b_ref[...]
    h = jnp.where(y >= 0.0, y, a_ref[...] * y)
    o_ref[...] = jnp.dot(h, w_ref[...], preferred_element_type=jnp.float32) * dis


def _k3_body(p_ref, hs_ref, degp_ref, b_ref, a_ref, o_ref):
    y = (p_ref[0] + p_ref[1] + hs_ref[...]) * _dis_of(degp_ref) + b_ref[...]
    o_ref[...] = jnp.where(y >= 0.0, y, a_ref[...] * y)


_row_spec = pl.BlockSpec((_RB, C), lambda i: (i, 0))
_w_spec = pl.BlockSpec((C, C), lambda i: (0, 0))
_vec_spec = pl.BlockSpec((1, C), lambda i: (0, 0))
_degp_spec = pl.BlockSpec((NC, _RB, C), lambda i: (0, i, 0))
_p_spec = pl.BlockSpec((NC, _RB, C), lambda i: (0, i, 0))
_out_shape = jax.ShapeDtypeStruct((N, C), jnp.float32)

_mm = pl.pallas_call(
    _mm_body,
    grid=(_GRID,),
    in_specs=[_row_spec, _w_spec],
    out_specs=_row_spec,
    out_shape=_out_shape,
)

_scale = pl.pallas_call(
    _scale_body,
    grid=(_GRID,),
    in_specs=[_row_spec, _degp_spec],
    out_specs=_row_spec,
    out_shape=_out_shape,
)

_k2 = pl.pallas_call(
    _k2_body,
    grid=(_GRID,),
    in_specs=[_p_spec, _row_spec, _degp_spec, _vec_spec, _vec_spec, _w_spec],
    out_specs=_row_spec,
    out_shape=_out_shape,
)

_k3 = pl.pallas_call(
    _k3_body,
    grid=(_GRID,),
    in_specs=[_p_spec, _row_spec, _degp_spec, _vec_spec, _vec_spec],
    out_specs=_row_spec,
    out_shape=_out_shape,
)


@jax.jit
def kernel(x, edge_index, W1, b1, alpha1, W2, b2, alpha2):
    src = edge_index[0].astype(jnp.int32)
    dst = edge_index[1].astype(jnp.int32)
    npad = E_PAD - E
    src = jnp.concatenate([src, jnp.zeros((npad,), jnp.int32)])
    dst = jnp.concatenate([dst, jnp.full((npad,), N, jnp.int32)])
    src_r = src.reshape(NW, KG, BG)
    dst_r = dst.reshape(NW, K, B)
    dst_g = dst.reshape(NW, KG, BG)

    zeros_acc = jnp.zeros((N_PAD, C), jnp.float32)
    b1r = b1.reshape(1, C)
    b2r = b2.reshape(1, C)
    a1r = alpha1.reshape(1, C)
    a2r = alpha2.reshape(1, C)

    h1 = _mm(x, W1)
    degp = _deg(dst_r, zeros_acc)
    hs1 = _scale(h1, degp)
    p1 = _agg(hs1, src_r, dst_g, zeros_acc)
    hs2 = _k2(p1, hs1, degp, b1r, a1r, W2)
    p2 = _agg(hs2, src_r, dst_g, zeros_acc)
    return _k3(p2, hs2, degp, b2r, a2r)
